# Initial kernel scaffold; baseline (speedup 1.0000x reference)
import jax
import jax.numpy as jnp
from jax.experimental import pallas as pl


def kernel(x, v, Wq, Wk, Wu, W1, b1, W2, b2, W3, b3, gamma, beta):
    raise NotImplementedError("write your pallas kernel here")


# speedup vs baseline: 4.4633x; 4.4633x over previous
import functools
import math

import jax
import jax.numpy as jnp
from jax import lax
from jax.experimental import pallas as pl

EPS = 1e-6
K_NN = 16
QBLK = 256
PBLK = 512
HIGH = lax.Precision.HIGHEST


def _proj_kernel(vf_ref, gq_ref, gk_ref, gu_ref, s_ref,
                 qf_ref, kf_ref, uf_ref, qn_ref, kn_ref):
    vf = vf_ref[...]
    s = s_ref[...]
    qf = lax.dot_general(vf, gq_ref[...], (((1,), (0,)), ((), ())),
                         precision=HIGH, preferred_element_type=jnp.float32)
    kf = lax.dot_general(vf, gk_ref[...], (((1,), (0,)), ((), ())),
                         precision=HIGH, preferred_element_type=jnp.float32)
    uf = lax.dot_general(vf, gu_ref[...], (((1,), (0,)), ((), ())),
                         precision=HIGH, preferred_element_type=jnp.float32)
    qf_ref[...] = qf
    kf_ref[...] = kf
    uf_ref[...] = uf
    qch = lax.dot_general(qf * qf, s, (((1,), (0,)), ((), ())),
                          precision=HIGH, preferred_element_type=jnp.float32)
    kch = lax.dot_general(kf * kf, s, (((1,), (0,)), ((), ())),
                          precision=HIGH, preferred_element_type=jnp.float32)
    qn_ref[...] = jnp.sum(jnp.sqrt(qch), axis=1, keepdims=True) * (1.0 / 64.0)
    kn_ref[...] = jnp.sum(jnp.sqrt(kch), axis=1, keepdims=True) * (1.0 / 64.0)


def _silu(x):
    return x * jax.nn.sigmoid(x)


def _main_kernel(xa_ref, xbt_ref, x2q_ref, x2r_ref,
                 qf_ref, kf_ref, uf_ref, qn_ref, knr_ref,
                 w1t_ref, b1_ref, w2t_ref, b2_ref, w3t_ref, b3_ref,
                 gamma_ref, beta_ref, s_ref, se_ref, out_ref, *, n_pts):
    i = pl.program_id(0)
    xab = xa_ref[...]
    xbt = xbt_ref[...]
    x2q = x2q_ref[...]
    x2r = x2r_ref[...]
    qfb = qf_ref[...]
    kfb = kf_ref[...]
    qn = qn_ref[...]
    knr = knr_ref[...]

    g = lax.dot_general(xab, xbt, (((1,), (0,)), ((), ())),
                        preferred_element_type=jnp.float32)
    d2 = (x2q + x2r) - 2.0 * g
    d2 = jnp.maximum(d2, 0.0)
    gh = lax.dot_general(xab, xbt, (((1,), (0,)), ((), ())),
                         precision=HIGH, preferred_element_type=jnp.float32)
    d2h = (x2q + x2r) - 2.0 * gh
    col = lax.broadcasted_iota(jnp.int32, (QBLK, n_pts), 1)
    rowg = lax.broadcasted_iota(jnp.int32, (QBLK, n_pts), 0) + i * QBLK
    d2 = jnp.where(col == rowg, jnp.inf, d2)

    di = lax.bitcast_convert_type(d2, jnp.int32)
    p0 = (di & jnp.int32(-4096)) | col

    qk = lax.dot_general(qfb, kfb, (((1,), (1,)), ((), ())),
                         precision=HIGH, preferred_element_type=jnp.float32)

    w1t = w1t_ref[...]
    b1 = b1_ref[...]
    w2t = w2t_ref[...]
    b2 = b2_ref[...]
    w3t = w3t_ref[...]
    b3 = b3_ref[...]
    big = jnp.int32(0x7FFFFFFF)

    def body(_, carry):
        p, wacc, se = carry
        m = jnp.min(p, axis=1, keepdims=True)
        oh = p == m
        p = jnp.where(oh, big, p)
        d2j = jnp.sum(jnp.where(oh, d2h, 0.0), axis=1, keepdims=True)
        dist = jnp.sqrt(jnp.maximum(d2j, 0.0))
        dotj = jnp.sum(jnp.where(oh, qk, 0.0), axis=1, keepdims=True) * (1.0 / 64.0)
        knj = jnp.sum(jnp.where(oh, knr, 0.0), axis=1, keepdims=True)
        h1 = _silu(qn * w1t[0:1, :] + knj * w1t[1:2, :]
                   + dotj * w1t[2:3, :] + dist * w1t[3:4, :] + b1)
        h2 = _silu(lax.dot_general(h1, w2t, (((1,), (0,)), ((), ())),
                                   precision=HIGH,
                                   preferred_element_type=jnp.float32) + b2)
        lg = lax.dot_general(h2, w3t, (((1,), (0,)), ((), ())),
                             precision=HIGH,
                             preferred_element_type=jnp.float32) + b3
        e = jnp.exp(jnp.clip(lg, -10.0, 10.0))
        return p, jnp.where(oh, e, wacc), se + e

    _, wacc, se = lax.fori_loop(
        0, K_NN,
        body,
        (p0, jnp.zeros((QBLK, n_pts), jnp.float32),
         jnp.zeros((QBLK, 1), jnp.float32)))

    msg = lax.dot_general(wacc, uf_ref[...], (((1,), (0,)), ((), ())),
                          precision=HIGH, preferred_element_type=jnp.float32)
    outb = qfb + 0.5 * (msg / se)

    chsq = lax.dot_general(outb * outb, s_ref[...], (((1,), (0,)), ((), ())),
                           precision=HIGH, preferred_element_type=jnp.float32)
    norm = jnp.maximum(jnp.sqrt(chsq), EPS)
    mean = jnp.mean(norm, axis=1, keepdims=True)
    cen = norm - mean
    std = jnp.maximum(jnp.sqrt(jnp.sum(cen * cen, axis=1, keepdims=True)
                               * (1.0 / 63.0)), EPS)
    ns = (cen / std) * gamma_ref[...] + beta_ref[...]
    mns = jnp.maximum(ns, EPS)
    fac = (mns / norm) * jnp.minimum(50.0 / mns, 1.0)
    face = lax.dot_general(fac, se_ref[...], (((1,), (0,)), ((), ())),
                           precision=HIGH, preferred_element_type=jnp.float32)
    out_ref[...] = outb * face


def kernel(x, v, Wq, Wk, Wu, W1, b1, W2, b2, W3, b3, gamma, beta):
    B, N = x.shape[0], x.shape[1]
    C = v.shape[2]
    K3 = 3 * C

    eye3 = jnp.eye(3, dtype=jnp.float32)
    gq = jnp.kron(Wq.T, eye3)
    gk = jnp.kron(Wk.T, eye3)
    gu = jnp.kron(Wu.T, eye3)
    s = jnp.kron(jnp.eye(C, dtype=jnp.float32), jnp.ones((3, 1), jnp.float32))
    se = s.T

    vf = v.reshape(B * N, K3)
    x2 = jnp.sum(x * x, axis=-1, keepdims=True)
    x2r = x2.reshape(B, 1, N)
    zero = jnp.zeros((B, N, 5), jnp.float32)
    xa = jnp.concatenate([x, zero], axis=-1)
    xbt = xa.transpose(0, 2, 1)

    nblk = (B * N) // PBLK
    qf, kf, uf, qn, kn = pl.pallas_call(
        _proj_kernel,
        grid=(nblk,),
        in_specs=[
            pl.BlockSpec((PBLK, K3), lambda j: (j, 0)),
            pl.BlockSpec((K3, K3), lambda j: (0, 0)),
            pl.BlockSpec((K3, K3), lambda j: (0, 0)),
            pl.BlockSpec((K3, K3), lambda j: (0, 0)),
            pl.BlockSpec((K3, C), lambda j: (0, 0)),
        ],
        out_specs=[
            pl.BlockSpec((PBLK, K3), lambda j: (j, 0)),
            pl.BlockSpec((PBLK, K3), lambda j: (j, 0)),
            pl.BlockSpec((PBLK, K3), lambda j: (j, 0)),
            pl.BlockSpec((PBLK, 1), lambda j: (j, 0)),
            pl.BlockSpec((PBLK, 1), lambda j: (j, 0)),
        ],
        out_shape=[
            jax.ShapeDtypeStruct((B * N, K3), jnp.float32),
            jax.ShapeDtypeStruct((B * N, K3), jnp.float32),
            jax.ShapeDtypeStruct((B * N, K3), jnp.float32),
            jax.ShapeDtypeStruct((B * N, 1), jnp.float32),
            jax.ShapeDtypeStruct((B * N, 1), jnp.float32),
        ],
    )(vf, gq, gk, gu, s)

    qf3 = qf.reshape(B, N, K3)
    kf3 = kf.reshape(B, N, K3)
    uf3 = uf.reshape(B, N, K3)
    qn3 = qn.reshape(B, N, 1)
    knr = kn.reshape(B, 1, N)

    w1t = W1.T
    w2t = W2.T
    w3t = W3.T
    b1r = b1.reshape(1, -1)
    b2r = b2.reshape(1, -1)
    b3r = b3.reshape(1, 1)
    gammar = gamma.reshape(1, C)
    betar = beta.reshape(1, C)

    main = pl.pallas_call(
        functools.partial(_main_kernel, n_pts=N),
        grid=(N // QBLK,),
        in_specs=[
            pl.BlockSpec((QBLK, 8), lambda i: (i, 0)),
            pl.BlockSpec((8, N), lambda i: (0, 0)),
            pl.BlockSpec((QBLK, 1), lambda i: (i, 0)),
            pl.BlockSpec((1, N), lambda i: (0, 0)),
            pl.BlockSpec((QBLK, K3), lambda i: (i, 0)),
            pl.BlockSpec((N, K3), lambda i: (0, 0)),
            pl.BlockSpec((N, K3), lambda i: (0, 0)),
            pl.BlockSpec((QBLK, 1), lambda i: (i, 0)),
            pl.BlockSpec((1, N), lambda i: (0, 0)),
            pl.BlockSpec((4, 32), lambda i: (0, 0)),
            pl.BlockSpec((1, 32), lambda i: (0, 0)),
            pl.BlockSpec((32, 32), lambda i: (0, 0)),
            pl.BlockSpec((1, 32), lambda i: (0, 0)),
            pl.BlockSpec((32, 1), lambda i: (0, 0)),
            pl.BlockSpec((1, 1), lambda i: (0, 0)),
            pl.BlockSpec((1, C), lambda i: (0, 0)),
            pl.BlockSpec((1, C), lambda i: (0, 0)),
            pl.BlockSpec((K3, C), lambda i: (0, 0)),
            pl.BlockSpec((C, K3), lambda i: (0, 0)),
        ],
        out_specs=pl.BlockSpec((QBLK, K3), lambda i: (i, 0)),
        out_shape=jax.ShapeDtypeStruct((N, K3), jnp.float32),
    )

    outs = []
    for b in range(B):
        outs.append(main(xa[b], xbt[b], x2[b], x2r[b],
                         qf3[b], kf3[b], uf3[b], qn3[b],
                         knr[b], w1t, b1r, w2t, b2r, w3t, b3r,
                         gammar, betar, s, se))
    out = jnp.stack(outs, axis=0)
    return out.reshape(B, N, C, 3)
